# baseline (speedup 1.0000x reference)
import functools

import jax
import jax.numpy as jnp
from jax import lax
from jax.experimental import pallas as pl
from jax.experimental.pallas import tpu as pltpu
from jax.experimental.pallas import tpu_sc as plsc

_N = 10000
_E = 320000
_D = 128
_NC = 2
_NS = 16
_NW = _NC * _NS
_CHUNK = 64
_EPW = 10240
_E_PAD = _EPW * _NW
_N_PAD = 10240
_RPT = _N_PAD // _NS

_NBUF = 4
_CPW = _EPW // _CHUNK
_NGRP = _CPW // _NBUF
_CPW0 = 248
_CPW1 = 2 * _CPW - _CPW0
_ZROWS = 64
_ZBASE = 10048
_SHIFT = 16384


def _unpack(packed_row, b, src_u, dst_u):
    for g in range(_CHUNK // 16):
        v = packed_row[pl.ds(g * 16, 16)]
        src_u[b][pl.ds(g * 16, 16)] = lax.rem(v, _SHIFT)
        dst_u[b][pl.ds(g * 16, 16)] = lax.div(v, _SHIFT)


def _make_agg():
    out_type = jax.ShapeDtypeStruct((_NC, _N_PAD, _D), jnp.float32)
    scratch = [
        pltpu.VMEM((_NBUF, _CHUNK), jnp.int32),
        pltpu.VMEM((_NBUF, _CHUNK), jnp.int32),
        pltpu.VMEM_SHARED((_N_PAD, _D), jnp.float32),
        pltpu.SemaphoreType.DMA,
        pltpu.SemaphoreType.DMA,
        pltpu.SemaphoreType.DMA,
    ]
    for _ in range(_NBUF):
        scratch.append(pltpu.VMEM((_CHUNK, _D), jnp.float32))
    for _ in range(_NBUF):
        scratch.append(pltpu.VMEM((_CHUNK,), jnp.int32))
    for _ in range(_NBUF):
        scratch.append(pltpu.VMEM((_CHUNK,), jnp.int32))
    for _ in range(2 * _NBUF):
        scratch.append(pltpu.SemaphoreType.DMA)
    mesh = plsc.VectorSubcoreMesh(core_axis_name="c", subcore_axis_name="s")

    @functools.partial(pl.kernel, mesh=mesh, out_type=out_type,
                       scratch_types=scratch,
                       compiler_params=pltpu.CompilerParams(
                           needs_layout_passes=False))
    def agg(x_hbm, idx_hbm, out_hbm, *rest):
        bank = rest[0:2]
        acc_sh, zsem = rest[2], rest[3]
        isem = rest[4:6]
        rows = rest[6:6 + _NBUF]
        src_u = rest[6 + _NBUF:6 + 2 * _NBUF]
        dst_u = rest[6 + 2 * _NBUF:6 + 3 * _NBUF]
        gsem = rest[6 + 3 * _NBUF:6 + 4 * _NBUF]
        ssem = rest[6 + 4 * _NBUF:6 + 5 * _NBUF]
        cid = lax.axis_index("c")
        sid = lax.axis_index("s")
        rbase = sid * _RPT
        ncpw = jnp.where(cid == 0, _CPW0, _CPW1)
        npair = ncpw // (2 * _NBUF)
        ngrp = ncpw // _NBUF
        irow = sid * (2 * _CPW) + cid * _CPW0

        zd = [pltpu.async_copy(x_hbm.at[pl.ds(_ZBASE, _ZROWS)],
                               acc_sh.at[pl.ds(rbase + k * _ZROWS, _ZROWS)],
                               zsem)
              for k in range(_RPT // _ZROWS)]
        pltpu.async_copy(idx_hbm.at[pl.ds(irow, _NBUF)], bank[0], isem[0])
        pltpu.async_copy(idx_hbm.at[pl.ds(irow + _NBUF, _NBUF)],
                         bank[1], isem[1])
        for d in zd:
            d.wait()
        plsc.subcore_barrier()

        def do_group(g, p, fire_next):
            pltpu.make_async_copy(idx_hbm.at[pl.ds(0, _NBUF)],
                                  bank[p], isem[p]).wait()
            for b in range(_NBUF):
                _unpack(bank[p].at[b], b, src_u, dst_u)
            if fire_next:
                pltpu.async_copy(
                    idx_hbm.at[pl.ds(irow + (g + 2) * _NBUF, _NBUF)],
                    bank[p], isem[p])
            gd = []
            for b in range(_NBUF):
                gd.append(pltpu.async_copy(
                    x_hbm.at[src_u[b]], rows[b], gsem[b]))
            sd = []
            for b in range(_NBUF):
                gd[b].wait()
                sd.append(pltpu.async_copy(
                    rows[b], acc_sh.at[dst_u[b]], ssem[b], add=True))
            for b in range(_NBUF):
                sd[b].wait()

        def pair(j2, _):
            do_group(2 * j2, 0, True)
            do_group(2 * j2 + 1, 1, True)
            return 0
        lax.fori_loop(0, npair - 1, pair, 0)
        do_group(ngrp - 2, 0, False)
        do_group(ngrp - 1, 1, False)
        plsc.subcore_barrier()

        pltpu.sync_copy(acc_sh.at[pl.ds(rbase, _RPT)],
                        out_hbm.at[cid, pl.ds(rbase, _RPT)])

    return agg


_agg = _make_agg()


@functools.partial(
    pl.kernel,
    mesh=plsc.VectorSubcoreMesh(core_axis_name="c", subcore_axis_name="s"),
    out_type=jax.ShapeDtypeStruct((_NC, _NS, _N_PAD), jnp.float32),
    scratch_types=[
        pltpu.VMEM((_CPW, _CHUNK), jnp.int32),
        pltpu.VMEM((_N_PAD,), jnp.float32),
        pltpu.SemaphoreType.DMA,
    ],
    compiler_params=pltpu.CompilerParams(needs_layout_passes=False),
)
def _deg_kernel(idx_hbm, deg_hbm, packed, deg_v, isem):
    cid = lax.axis_index("c")
    sid = lax.axis_index("s")
    wid = sid * _NC + cid
    d_idx = pltpu.async_copy(idx_hbm.at[pl.ds(wid * _CPW, _CPW)],
                             packed, isem)
    z16 = jnp.zeros((16,), jnp.float32)

    def zloop(k, _):
        deg_v[pl.ds(k * 16, 16)] = z16
        return 0
    lax.fori_loop(0, _N_PAD // 16, zloop, 0)
    d_idx.wait()
    ones16 = jnp.full((16,), 1.0, jnp.float32)

    def dloop(r, _):
        for g in range(_CHUNK // 16):
            v = packed[r, pl.ds(g * 16, 16)]
            plsc.addupdate_scatter(deg_v, [lax.div(v, _SHIFT)], ones16)
        return 0
    lax.fori_loop(0, _CPW, dloop, 0)
    pltpu.sync_copy(deg_v, deg_hbm.at[cid, sid])


_TCR = 640


def _tc_body(agg_ref, deg_ref, x_ref, w_ref, b_ref, g_ref, be_ref, out_ref):
    agg = agg_ref[0] + agg_ref[1] + x_ref[...]
    deg = jnp.sum(deg_ref[...], axis=0)
    h = agg / (deg + 1.0)[:, None]
    h = jnp.dot(h, w_ref[...], preferred_element_type=jnp.float32)
    h = h + b_ref[...]
    mu = jnp.mean(h, axis=1, keepdims=True)
    var = jnp.mean((h - mu) ** 2, axis=1, keepdims=True)
    h = (h - mu) * lax.rsqrt(var + 1e-5) * g_ref[...] + be_ref[...]
    h = jnp.where(h > 0, h, jnp.exp(jnp.minimum(h, 0.0)) - 1.0)
    rowid = (pl.program_id(0) * _TCR
             + lax.broadcasted_iota(jnp.int32, (_TCR, _D), 0))
    out_ref[...] = jnp.where(rowid < _N, h, 0.0)


@jax.jit
def _tc_layer(agg, deg_t, x, w, b, g, be):
    grid = _N_PAD // _TCR
    return pl.pallas_call(
        _tc_body,
        grid=(grid,),
        in_specs=[
            pl.BlockSpec((_NC, _TCR, _D), lambda i: (0, i, 0)),
            pl.BlockSpec((_NC * 16, _TCR), lambda i: (0, i)),
            pl.BlockSpec((_TCR, _D), lambda i: (i, 0)),
            pl.BlockSpec((_D, _D), lambda i: (0, 0)),
            pl.BlockSpec((1, _D), lambda i: (0, 0)),
            pl.BlockSpec((1, _D), lambda i: (0, 0)),
            pl.BlockSpec((1, _D), lambda i: (0, 0)),
        ],
        out_specs=pl.BlockSpec((_TCR, _D), lambda i: (i, 0)),
        out_shape=jax.ShapeDtypeStruct((_N_PAD, _D), jnp.float32),
    )(agg, deg_t, x, w, b, g, be)


_B_UID = 1024
_BPW = _B_UID // _NW


@functools.partial(
    pl.kernel,
    mesh=plsc.VectorSubcoreMesh(core_axis_name="c", subcore_axis_name="s"),
    out_type=jax.ShapeDtypeStruct((_B_UID, _D), jnp.float32),
    scratch_types=[
        pltpu.VMEM((_BPW,), jnp.int32),
        pltpu.VMEM((_BPW, _D), jnp.float32),
        pltpu.SemaphoreType.DMA,
    ],
)
def _uid_gather(feats_hbm, uid_hbm, out_hbm, idx_v, rows_v, sem):
    wid = lax.axis_index("s") * _NC + lax.axis_index("c")
    base = wid * _BPW
    pltpu.sync_copy(uid_hbm.at[pl.ds(base, _BPW)], idx_v)
    pltpu.async_copy(feats_hbm.at[idx_v], rows_v, sem).wait()
    pltpu.sync_copy(rows_v, out_hbm.at[pl.ds(base, _BPW)])


def kernel(uid, edge_index, embedding, W0, b0, gamma0, beta0,
           W1, b1, gamma1, beta1, W2, b2, gamma2, beta2):
    src = edge_index[0]
    dst = edge_index[1]
    pad = _E_PAD - _E
    padv = jnp.full((pad,), _N, jnp.int32)
    src_p = jnp.concatenate([src, padv])
    dst_p = jnp.concatenate([dst, padv])
    packed = (dst_p * _SHIFT + src_p).reshape(_NW * _CPW, _CHUNK)
    x = jnp.zeros((_N_PAD, _D), jnp.float32).at[:_N].set(embedding)

    deg = _deg_kernel(packed)
    deg_t = deg.reshape(_NC * _NS, _N_PAD)
    agg = _agg(x, packed)
    h = _tc_layer(agg, deg_t, x, W0, b0.reshape(1, _D),
                  gamma0.reshape(1, _D), beta0.reshape(1, _D))
    agg = _agg(h, packed)
    h = _tc_layer(agg, deg_t, h, W1, b1.reshape(1, _D),
                  gamma1.reshape(1, _D), beta1.reshape(1, _D))
    agg = _agg(h, packed)
    h = _tc_layer(agg, deg_t, h, W2, b2.reshape(1, _D),
                  gamma2.reshape(1, _D), beta2.reshape(1, _D))
    return _uid_gather(h, uid)


# speedup vs baseline: 1.0329x; 1.0329x over previous
import functools

import jax
import jax.numpy as jnp
from jax import lax
from jax.experimental import pallas as pl
from jax.experimental.pallas import tpu as pltpu
from jax.experimental.pallas import tpu_sc as plsc

_N = 10000
_E = 320000
_D = 128
_NC = 2
_NS = 16
_NW = _NC * _NS
_CHUNK = 64
_EPW = 10240
_E_PAD = _EPW * _NW
_N_PAD = 10240
_RPT = _N_PAD // _NS

_NBUF = 4
_CPW = _EPW // _CHUNK
_NGRP = _CPW // _NBUF
_CPW0 = 240
_CPW1 = 2 * _CPW - _CPW0
_ZROWS = 64
_ZBASE = 10048
_SHIFT = 16384


def _unpack(packed_row, b, src_u, dst_u):
    for g in range(_CHUNK // 16):
        v = packed_row[pl.ds(g * 16, 16)]
        src_u[b][pl.ds(g * 16, 16)] = lax.rem(v, _SHIFT)
        dst_u[b][pl.ds(g * 16, 16)] = lax.div(v, _SHIFT)


def _make_agg():
    out_type = jax.ShapeDtypeStruct((_NC, _N_PAD, _D), jnp.float32)
    scratch = [
        pltpu.VMEM((_NBUF, _CHUNK), jnp.int32),
        pltpu.VMEM((_NBUF, _CHUNK), jnp.int32),
        pltpu.VMEM_SHARED((_N_PAD, _D), jnp.float32),
        pltpu.SemaphoreType.DMA,
        pltpu.SemaphoreType.DMA,
        pltpu.SemaphoreType.DMA,
    ]
    for _ in range(_NBUF):
        scratch.append(pltpu.VMEM((_CHUNK, _D), jnp.float32))
    for _ in range(_NBUF):
        scratch.append(pltpu.VMEM((_CHUNK,), jnp.int32))
    for _ in range(_NBUF):
        scratch.append(pltpu.VMEM((_CHUNK,), jnp.int32))
    for _ in range(2 * _NBUF):
        scratch.append(pltpu.SemaphoreType.DMA)
    mesh = plsc.VectorSubcoreMesh(core_axis_name="c", subcore_axis_name="s")

    @functools.partial(pl.kernel, mesh=mesh, out_type=out_type,
                       scratch_types=scratch,
                       compiler_params=pltpu.CompilerParams(
                           needs_layout_passes=False))
    def agg(x_hbm, idx_hbm, out_hbm, *rest):
        bank = rest[0:2]
        acc_sh, zsem = rest[2], rest[3]
        isem = rest[4:6]
        rows = rest[6:6 + _NBUF]
        src_u = rest[6 + _NBUF:6 + 2 * _NBUF]
        dst_u = rest[6 + 2 * _NBUF:6 + 3 * _NBUF]
        gsem = rest[6 + 3 * _NBUF:6 + 4 * _NBUF]
        ssem = rest[6 + 4 * _NBUF:6 + 5 * _NBUF]
        cid = lax.axis_index("c")
        sid = lax.axis_index("s")
        rbase = sid * _RPT
        ncpw = jnp.where(cid == 0, _CPW0, _CPW1)
        npair = ncpw // (2 * _NBUF)
        ngrp = ncpw // _NBUF
        irow = sid * (2 * _CPW) + cid * _CPW0

        zd = [pltpu.async_copy(x_hbm.at[pl.ds(_ZBASE, _ZROWS)],
                               acc_sh.at[pl.ds(rbase + k * _ZROWS, _ZROWS)],
                               zsem)
              for k in range(_RPT // _ZROWS)]
        pltpu.async_copy(idx_hbm.at[pl.ds(irow, _NBUF)], bank[0], isem[0])
        pltpu.async_copy(idx_hbm.at[pl.ds(irow + _NBUF, _NBUF)],
                         bank[1], isem[1])
        for d in zd:
            d.wait()
        plsc.subcore_barrier()

        def do_group(g, p, fire_next):
            pltpu.make_async_copy(idx_hbm.at[pl.ds(0, _NBUF)],
                                  bank[p], isem[p]).wait()
            for b in range(_NBUF):
                _unpack(bank[p].at[b], b, src_u, dst_u)
            if fire_next:
                pltpu.async_copy(
                    idx_hbm.at[pl.ds(irow + (g + 2) * _NBUF, _NBUF)],
                    bank[p], isem[p])
            gd = []
            for b in range(_NBUF):
                gd.append(pltpu.async_copy(
                    x_hbm.at[src_u[b]], rows[b], gsem[b]))
            sd = []
            for b in range(_NBUF):
                gd[b].wait()
                sd.append(pltpu.async_copy(
                    rows[b], acc_sh.at[dst_u[b]], ssem[b], add=True))
            for b in range(_NBUF):
                sd[b].wait()

        def pair(j2, _):
            do_group(2 * j2, 0, True)
            do_group(2 * j2 + 1, 1, True)
            return 0
        lax.fori_loop(0, npair - 1, pair, 0)
        do_group(ngrp - 2, 0, False)
        do_group(ngrp - 1, 1, False)
        plsc.subcore_barrier()

        pltpu.sync_copy(acc_sh.at[pl.ds(rbase, _RPT)],
                        out_hbm.at[cid, pl.ds(rbase, _RPT)])

    return agg


_agg = _make_agg()


@functools.partial(
    pl.kernel,
    mesh=plsc.VectorSubcoreMesh(core_axis_name="c", subcore_axis_name="s"),
    out_type=jax.ShapeDtypeStruct((_NC, _NS, _N_PAD), jnp.float32),
    scratch_types=[
        pltpu.VMEM((_CPW, _CHUNK), jnp.int32),
        pltpu.VMEM((_N_PAD,), jnp.float32),
        pltpu.SemaphoreType.DMA,
    ],
    compiler_params=pltpu.CompilerParams(needs_layout_passes=False),
)
def _deg_kernel(idx_hbm, deg_hbm, packed, deg_v, isem):
    cid = lax.axis_index("c")
    sid = lax.axis_index("s")
    wid = sid * _NC + cid
    d_idx = pltpu.async_copy(idx_hbm.at[pl.ds(wid * _CPW, _CPW)],
                             packed, isem)
    z16 = jnp.zeros((16,), jnp.float32)

    def zloop(k, _):
        deg_v[pl.ds(k * 16, 16)] = z16
        return 0
    lax.fori_loop(0, _N_PAD // 16, zloop, 0)
    d_idx.wait()
    ones16 = jnp.full((16,), 1.0, jnp.float32)

    def dloop(r, _):
        for g in range(_CHUNK // 16):
            v = packed[r, pl.ds(g * 16, 16)]
            plsc.addupdate_scatter(deg_v, [lax.div(v, _SHIFT)], ones16)
        return 0
    lax.fori_loop(0, _CPW, dloop, 0)
    pltpu.sync_copy(deg_v, deg_hbm.at[cid, sid])


_TCR = 640


def _tc_body(agg_ref, deg_ref, x_ref, w_ref, b_ref, g_ref, be_ref, out_ref):
    agg = agg_ref[0] + agg_ref[1] + x_ref[...]
    deg = jnp.sum(deg_ref[...], axis=0)
    h = agg / (deg + 1.0)[:, None]
    h = jnp.dot(h, w_ref[...], preferred_element_type=jnp.float32)
    h = h + b_ref[...]
    mu = jnp.mean(h, axis=1, keepdims=True)
    var = jnp.mean((h - mu) ** 2, axis=1, keepdims=True)
    h = (h - mu) * lax.rsqrt(var + 1e-5) * g_ref[...] + be_ref[...]
    h = jnp.where(h > 0, h, jnp.exp(jnp.minimum(h, 0.0)) - 1.0)
    rowid = (pl.program_id(0) * _TCR
             + lax.broadcasted_iota(jnp.int32, (_TCR, _D), 0))
    out_ref[...] = jnp.where(rowid < _N, h, 0.0)


@jax.jit
def _tc_layer(agg, deg_t, x, w, b, g, be):
    grid = _N_PAD // _TCR
    return pl.pallas_call(
        _tc_body,
        grid=(grid,),
        in_specs=[
            pl.BlockSpec((_NC, _TCR, _D), lambda i: (0, i, 0)),
            pl.BlockSpec((_NC * 16, _TCR), lambda i: (0, i)),
            pl.BlockSpec((_TCR, _D), lambda i: (i, 0)),
            pl.BlockSpec((_D, _D), lambda i: (0, 0)),
            pl.BlockSpec((1, _D), lambda i: (0, 0)),
            pl.BlockSpec((1, _D), lambda i: (0, 0)),
            pl.BlockSpec((1, _D), lambda i: (0, 0)),
        ],
        out_specs=pl.BlockSpec((_TCR, _D), lambda i: (i, 0)),
        out_shape=jax.ShapeDtypeStruct((_N_PAD, _D), jnp.float32),
    )(agg, deg_t, x, w, b, g, be)


_B_UID = 1024
_BPW = _B_UID // _NW


@functools.partial(
    pl.kernel,
    mesh=plsc.VectorSubcoreMesh(core_axis_name="c", subcore_axis_name="s"),
    out_type=jax.ShapeDtypeStruct((_B_UID, _D), jnp.float32),
    scratch_types=[
        pltpu.VMEM((_BPW,), jnp.int32),
        pltpu.VMEM((_BPW, _D), jnp.float32),
        pltpu.SemaphoreType.DMA,
    ],
)
def _uid_gather(feats_hbm, uid_hbm, out_hbm, idx_v, rows_v, sem):
    wid = lax.axis_index("s") * _NC + lax.axis_index("c")
    base = wid * _BPW
    pltpu.sync_copy(uid_hbm.at[pl.ds(base, _BPW)], idx_v)
    pltpu.async_copy(feats_hbm.at[idx_v], rows_v, sem).wait()
    pltpu.sync_copy(rows_v, out_hbm.at[pl.ds(base, _BPW)])


def kernel(uid, edge_index, embedding, W0, b0, gamma0, beta0,
           W1, b1, gamma1, beta1, W2, b2, gamma2, beta2):
    src = edge_index[0]
    dst = edge_index[1]
    pad = _E_PAD - _E
    padv = jnp.full((pad,), _N, jnp.int32)
    src_p = jnp.concatenate([src, padv])
    dst_p = jnp.concatenate([dst, padv])
    packed = (dst_p * _SHIFT + src_p).reshape(_NW * _CPW, _CHUNK)
    x = jnp.zeros((_N_PAD, _D), jnp.float32).at[:_N].set(embedding)

    deg = _deg_kernel(packed)
    deg_t = deg.reshape(_NC * _NS, _N_PAD)
    agg = _agg(x, packed)
    h = _tc_layer(agg, deg_t, x, W0, b0.reshape(1, _D),
                  gamma0.reshape(1, _D), beta0.reshape(1, _D))
    agg = _agg(h, packed)
    h = _tc_layer(agg, deg_t, h, W1, b1.reshape(1, _D),
                  gamma1.reshape(1, _D), beta1.reshape(1, _D))
    agg = _agg(h, packed)
    h = _tc_layer(agg, deg_t, h, W2, b2.reshape(1, _D),
                  gamma2.reshape(1, _D), beta2.reshape(1, _D))
    return _uid_gather(h, uid)


# speedup vs baseline: 1.0776x; 1.0432x over previous
import functools

import jax
import jax.numpy as jnp
from jax import lax
from jax.experimental import pallas as pl
from jax.experimental.pallas import tpu as pltpu
from jax.experimental.pallas import tpu_sc as plsc

_N = 10000
_E = 320000
_D = 128
_NC = 2
_NS = 16
_NW = _NC * _NS
_CHUNK = 128
_EPW = 10240
_E_PAD = _EPW * _NW
_N_PAD = 10240
_RPT = _N_PAD // _NS

_NBUF = 2
_CPW = _EPW // _CHUNK
_NGRP = _CPW // _NBUF
_CPW0 = 120
_CPW1 = 2 * _CPW - _CPW0
_ZROWS = 64
_ZBASE = 10048
_SHIFT = 16384


def _unpack(packed_row, b, src_u, dst_u):
    for g in range(_CHUNK // 16):
        v = packed_row[pl.ds(g * 16, 16)]
        src_u[b][pl.ds(g * 16, 16)] = lax.rem(v, _SHIFT)
        dst_u[b][pl.ds(g * 16, 16)] = lax.div(v, _SHIFT)


def _make_agg():
    out_type = jax.ShapeDtypeStruct((_NC, _N_PAD, _D), jnp.float32)
    scratch = [
        pltpu.VMEM((_NBUF, _CHUNK), jnp.int32),
        pltpu.VMEM((_NBUF, _CHUNK), jnp.int32),
        pltpu.VMEM_SHARED((_N_PAD, _D), jnp.float32),
        pltpu.SemaphoreType.DMA,
        pltpu.SemaphoreType.DMA,
        pltpu.SemaphoreType.DMA,
    ]
    for _ in range(_NBUF):
        scratch.append(pltpu.VMEM((_CHUNK, _D), jnp.float32))
    for _ in range(_NBUF):
        scratch.append(pltpu.VMEM((_CHUNK,), jnp.int32))
    for _ in range(_NBUF):
        scratch.append(pltpu.VMEM((_CHUNK,), jnp.int32))
    for _ in range(2 * _NBUF):
        scratch.append(pltpu.SemaphoreType.DMA)
    mesh = plsc.VectorSubcoreMesh(core_axis_name="c", subcore_axis_name="s")

    @functools.partial(pl.kernel, mesh=mesh, out_type=out_type,
                       scratch_types=scratch,
                       compiler_params=pltpu.CompilerParams(
                           needs_layout_passes=False))
    def agg(x_hbm, idx_hbm, out_hbm, *rest):
        bank = rest[0:2]
        acc_sh, zsem = rest[2], rest[3]
        isem = rest[4:6]
        rows = rest[6:6 + _NBUF]
        src_u = rest[6 + _NBUF:6 + 2 * _NBUF]
        dst_u = rest[6 + 2 * _NBUF:6 + 3 * _NBUF]
        gsem = rest[6 + 3 * _NBUF:6 + 4 * _NBUF]
        ssem = rest[6 + 4 * _NBUF:6 + 5 * _NBUF]
        cid = lax.axis_index("c")
        sid = lax.axis_index("s")
        rbase = sid * _RPT
        ncpw = jnp.where(cid == 0, _CPW0, _CPW1)
        npair = ncpw // (2 * _NBUF)
        ngrp = ncpw // _NBUF
        irow = sid * (2 * _CPW) + cid * _CPW0

        zd = [pltpu.async_copy(x_hbm.at[pl.ds(_ZBASE, _ZROWS)],
                               acc_sh.at[pl.ds(rbase + k * _ZROWS, _ZROWS)],
                               zsem)
              for k in range(_RPT // _ZROWS)]
        pltpu.async_copy(idx_hbm.at[pl.ds(irow, _NBUF)], bank[0], isem[0])
        pltpu.async_copy(idx_hbm.at[pl.ds(irow + _NBUF, _NBUF)],
                         bank[1], isem[1])
        for d in zd:
            d.wait()
        plsc.subcore_barrier()

        def do_group(g, p, fire_next):
            pltpu.make_async_copy(idx_hbm.at[pl.ds(0, _NBUF)],
                                  bank[p], isem[p]).wait()
            for b in range(_NBUF):
                _unpack(bank[p].at[b], b, src_u, dst_u)
            if fire_next:
                pltpu.async_copy(
                    idx_hbm.at[pl.ds(irow + (g + 2) * _NBUF, _NBUF)],
                    bank[p], isem[p])
            gd = []
            for b in range(_NBUF):
                gd.append(pltpu.async_copy(
                    x_hbm.at[src_u[b]], rows[b], gsem[b]))
            sd = []
            for b in range(_NBUF):
                gd[b].wait()
                sd.append(pltpu.async_copy(
                    rows[b], acc_sh.at[dst_u[b]], ssem[b], add=True))
            for b in range(_NBUF):
                sd[b].wait()

        def pair(j2, _):
            do_group(2 * j2, 0, True)
            do_group(2 * j2 + 1, 1, True)
            return 0
        lax.fori_loop(0, npair - 1, pair, 0)
        do_group(ngrp - 2, 0, False)
        do_group(ngrp - 1, 1, False)
        plsc.subcore_barrier()

        pltpu.sync_copy(acc_sh.at[pl.ds(rbase, _RPT)],
                        out_hbm.at[cid, pl.ds(rbase, _RPT)])

    return agg


_agg = _make_agg()


@functools.partial(
    pl.kernel,
    mesh=plsc.VectorSubcoreMesh(core_axis_name="c", subcore_axis_name="s"),
    out_type=jax.ShapeDtypeStruct((_NC, _NS, _N_PAD), jnp.float32),
    scratch_types=[
        pltpu.VMEM((_CPW, _CHUNK), jnp.int32),
        pltpu.VMEM((_N_PAD,), jnp.float32),
        pltpu.SemaphoreType.DMA,
    ],
    compiler_params=pltpu.CompilerParams(needs_layout_passes=False),
)
def _deg_kernel(idx_hbm, deg_hbm, packed, deg_v, isem):
    cid = lax.axis_index("c")
    sid = lax.axis_index("s")
    wid = sid * _NC + cid
    d_idx = pltpu.async_copy(idx_hbm.at[pl.ds(wid * _CPW, _CPW)],
                             packed, isem)
    z16 = jnp.zeros((16,), jnp.float32)

    def zloop(k, _):
        deg_v[pl.ds(k * 16, 16)] = z16
        return 0
    lax.fori_loop(0, _N_PAD // 16, zloop, 0)
    d_idx.wait()
    ones16 = jnp.full((16,), 1.0, jnp.float32)

    def dloop(r, _):
        for g in range(_CHUNK // 16):
            v = packed[r, pl.ds(g * 16, 16)]
            plsc.addupdate_scatter(deg_v, [lax.div(v, _SHIFT)], ones16)
        return 0
    lax.fori_loop(0, _CPW, dloop, 0)
    pltpu.sync_copy(deg_v, deg_hbm.at[cid, sid])


_TCR = 640


def _tc_body(agg_ref, deg_ref, x_ref, w_ref, b_ref, g_ref, be_ref, out_ref):
    agg = agg_ref[0] + agg_ref[1] + x_ref[...]
    deg = jnp.sum(deg_ref[...], axis=0)
    h = agg / (deg + 1.0)[:, None]
    h = jnp.dot(h, w_ref[...], preferred_element_type=jnp.float32)
    h = h + b_ref[...]
    mu = jnp.mean(h, axis=1, keepdims=True)
    var = jnp.mean((h - mu) ** 2, axis=1, keepdims=True)
    h = (h - mu) * lax.rsqrt(var + 1e-5) * g_ref[...] + be_ref[...]
    h = jnp.where(h > 0, h, jnp.exp(jnp.minimum(h, 0.0)) - 1.0)
    rowid = (pl.program_id(0) * _TCR
             + lax.broadcasted_iota(jnp.int32, (_TCR, _D), 0))
    out_ref[...] = jnp.where(rowid < _N, h, 0.0)


@jax.jit
def _tc_layer(agg, deg_t, x, w, b, g, be):
    grid = _N_PAD // _TCR
    return pl.pallas_call(
        _tc_body,
        grid=(grid,),
        in_specs=[
            pl.BlockSpec((_NC, _TCR, _D), lambda i: (0, i, 0)),
            pl.BlockSpec((_NC * 16, _TCR), lambda i: (0, i)),
            pl.BlockSpec((_TCR, _D), lambda i: (i, 0)),
            pl.BlockSpec((_D, _D), lambda i: (0, 0)),
            pl.BlockSpec((1, _D), lambda i: (0, 0)),
            pl.BlockSpec((1, _D), lambda i: (0, 0)),
            pl.BlockSpec((1, _D), lambda i: (0, 0)),
        ],
        out_specs=pl.BlockSpec((_TCR, _D), lambda i: (i, 0)),
        out_shape=jax.ShapeDtypeStruct((_N_PAD, _D), jnp.float32),
    )(agg, deg_t, x, w, b, g, be)


_B_UID = 1024
_BPW = _B_UID // _NW


@functools.partial(
    pl.kernel,
    mesh=plsc.VectorSubcoreMesh(core_axis_name="c", subcore_axis_name="s"),
    out_type=jax.ShapeDtypeStruct((_B_UID, _D), jnp.float32),
    scratch_types=[
        pltpu.VMEM((_BPW,), jnp.int32),
        pltpu.VMEM((_BPW, _D), jnp.float32),
        pltpu.SemaphoreType.DMA,
    ],
)
def _uid_gather(feats_hbm, uid_hbm, out_hbm, idx_v, rows_v, sem):
    wid = lax.axis_index("s") * _NC + lax.axis_index("c")
    base = wid * _BPW
    pltpu.sync_copy(uid_hbm.at[pl.ds(base, _BPW)], idx_v)
    pltpu.async_copy(feats_hbm.at[idx_v], rows_v, sem).wait()
    pltpu.sync_copy(rows_v, out_hbm.at[pl.ds(base, _BPW)])


def kernel(uid, edge_index, embedding, W0, b0, gamma0, beta0,
           W1, b1, gamma1, beta1, W2, b2, gamma2, beta2):
    src = edge_index[0]
    dst = edge_index[1]
    pad = _E_PAD - _E
    padv = jnp.full((pad,), _N, jnp.int32)
    src_p = jnp.concatenate([src, padv])
    dst_p = jnp.concatenate([dst, padv])
    packed = (dst_p * _SHIFT + src_p).reshape(_NW * _CPW, _CHUNK)
    x = jnp.zeros((_N_PAD, _D), jnp.float32).at[:_N].set(embedding)

    deg = _deg_kernel(packed)
    deg_t = deg.reshape(_NC * _NS, _N_PAD)
    agg = _agg(x, packed)
    h = _tc_layer(agg, deg_t, x, W0, b0.reshape(1, _D),
                  gamma0.reshape(1, _D), beta0.reshape(1, _D))
    agg = _agg(h, packed)
    h = _tc_layer(agg, deg_t, h, W1, b1.reshape(1, _D),
                  gamma1.reshape(1, _D), beta1.reshape(1, _D))
    agg = _agg(h, packed)
    h = _tc_layer(agg, deg_t, h, W2, b2.reshape(1, _D),
                  gamma2.reshape(1, _D), beta2.reshape(1, _D))
    return _uid_gather(h, uid)
